# baseline (device time: 32213 ns/iter reference)
import jax
import jax.numpy as jnp
from jax import lax
from jax.experimental import pallas as pl
from jax.experimental.pallas import tpu as pltpu

Z = 4
C = 8


def kernel(x, pi):
    _, m, n = x.shape
    q = m // 4
    rc = q // C

    def body(
        pi_ref, x_ref, out_ref, qsend, qbuf,
        sc_send, scz_recv, scax_recv, scay_recv, scb_recv,
        sc_sems,
        send_z, recv_z, send_ax, recv_ax,
        send_ay, recv_ay, send_b, recv_b,
    ):
        mx = lax.axis_index("x")
        my = lax.axis_index("y")
        mz = lax.axis_index("z")

        dst_z = pi_ref[mz]
        src_z = jnp.int32(0)
        for j in range(Z):
            src_z = jnp.where(pi_ref[j] == mz, jnp.int32(j), src_z)

        p = 2 * mx + my
        qx = 2 * (1 - mx) + my
        qy = 2 * mx + (1 - my)
        qd = 2 * (1 - mx) + (1 - my)
        xnbr = (1 - mx, my, mz)
        ynbr = (mx, 1 - my, mz)
        dnbr = (1 - mx, 1 - my, mz)

        barrier = pltpu.get_barrier_semaphore()
        for dev in [(mx, my, src_z), xnbr, ynbr, dnbr]:
            pl.semaphore_signal(
                barrier, inc=1, device_id=dev,
                device_id_type=pl.DeviceIdType.MESH,
            )
        pl.semaphore_wait(barrier, 4)

        def copy(src, dst, ssem, rsem, dev):
            return pltpu.make_async_remote_copy(
                src_ref=src, dst_ref=dst, send_sem=ssem, recv_sem=rsem,
                device_id=dev, device_id_type=pl.DeviceIdType.MESH,
            )

        absmax = jnp.maximum(
            jnp.max(jnp.abs(x_ref[:, pl.ds(p * q, q), :])), 1e-30
        )
        sc_send[...] = jnp.full((8, 128), absmax * (1.0 / 127.0), jnp.float32)
        rscz = copy(sc_send, scz_recv, sc_sems.at[0], sc_sems.at[1],
                    (mx, my, dst_z))
        rscz.start()

        def chunk(i, c):
            return qbuf.at[:, pl.ds(i * q + c * rc, rc), :]

        inv = 127.0 / absmax
        rz = []
        for c in range(C):
            rows = pl.ds(p * q + c * rc, rc)
            crows = pl.ds(c * rc, rc)
            xq = x_ref[:, rows, :] * inv
            qsend[:, crows, :] = jnp.clip(
                jnp.round(xq), -127.0, 127.0
            ).astype(jnp.int8)
            r = copy(
                qsend.at[:, crows, :], chunk(p, c),
                send_z.at[c], recv_z.at[c], (mx, my, dst_z),
            )
            r.start()
            rz.append(r)

        rscz.wait_recv()
        dq_p = scz_recv[0, 0]
        rscax = copy(scz_recv, scax_recv, sc_sems.at[2], sc_sems.at[3], xnbr)
        rscay = copy(scz_recv, scay_recv, sc_sems.at[4], sc_sems.at[5], ynbr)
        rscd = copy(scz_recv, scb_recv, sc_sems.at[6], sc_sems.at[7], dnbr)
        rscax.start()
        rscay.start()
        rscd.start()

        def dequant(i, c, dq):
            rows = pl.ds(i * q + c * rc, rc)
            out_ref[:, rows, :] = (
                qbuf[:, rows, :].astype(jnp.bfloat16) * dq.astype(jnp.bfloat16)
            )

        rax, ray, rd = [], [], []
        for c in range(C):
            rz[c].wait_recv()
            ra = copy(chunk(p, c), chunk(p, c), send_ax.at[c], recv_ax.at[c], xnbr)
            ry = copy(chunk(p, c), chunk(p, c), send_ay.at[c], recv_ay.at[c], ynbr)
            rr = copy(chunk(p, c), chunk(p, c), send_b.at[c], recv_b.at[c], dnbr)
            ra.start()
            ry.start()
            rr.start()
            rax.append(ra)
            ray.append(ry)
            rd.append(rr)
            dequant(p, c, dq_p)

        rscax.wait_recv()
        dq_qx = scax_recv[0, 0]
        rscay.wait_recv()
        dq_qy = scay_recv[0, 0]
        rscd.wait_recv()
        dq_qd = scb_recv[0, 0]

        for c in range(C):
            rax[c].wait_recv()
            dequant(qx, c, dq_qx)
            ray[c].wait_recv()
            dequant(qy, c, dq_qy)
            rd[c].wait_recv()
            dequant(qd, c, dq_qd)

        for r in [rscz, rscax, rscay, rscd] + rz + rax + ray + rd:
            r.wait_send()

    return pl.pallas_call(
        body,
        out_shape=jax.ShapeDtypeStruct((1, m, n), jnp.bfloat16),
        in_specs=[
            pl.BlockSpec(memory_space=pltpu.SMEM),
            pl.BlockSpec(memory_space=pltpu.VMEM),
        ],
        out_specs=pl.BlockSpec(memory_space=pltpu.VMEM),
        scratch_shapes=[
            pltpu.VMEM((1, q, n), jnp.int8),
            pltpu.VMEM((1, m, n), jnp.int8),
            pltpu.VMEM((8, 128), jnp.float32),
            pltpu.VMEM((8, 128), jnp.float32),
            pltpu.VMEM((8, 128), jnp.float32),
            pltpu.VMEM((8, 128), jnp.float32),
            pltpu.VMEM((8, 128), jnp.float32),
            pltpu.SemaphoreType.DMA((8,)),
            pltpu.SemaphoreType.DMA((C,)),
            pltpu.SemaphoreType.DMA((C,)),
            pltpu.SemaphoreType.DMA((C,)),
            pltpu.SemaphoreType.DMA((C,)),
            pltpu.SemaphoreType.DMA((C,)),
            pltpu.SemaphoreType.DMA((C,)),
            pltpu.SemaphoreType.DMA((C,)),
            pltpu.SemaphoreType.DMA((C,)),
        ],
        compiler_params=pltpu.CompilerParams(collective_id=0),
    )(pi, x)


# device time: 30354 ns/iter; 1.0612x vs baseline; 1.0612x over previous
import jax
import jax.numpy as jnp
from jax import lax
from jax.experimental import pallas as pl
from jax.experimental.pallas import tpu as pltpu

Z = 4
C = 16


def kernel(x, pi):
    _, m, n = x.shape
    q = m // 4
    rc = q // C

    def body(
        pi_ref, x_ref, out_ref, qsend, qbuf,
        sc_send, scz_recv, scax_recv, scay_recv, scb_recv,
        sc_sems,
        send_z, recv_z, send_ax, recv_ax,
        send_ay, recv_ay, send_b, recv_b,
    ):
        mx = lax.axis_index("x")
        my = lax.axis_index("y")
        mz = lax.axis_index("z")

        dst_z = pi_ref[mz]
        src_z = jnp.int32(0)
        for j in range(Z):
            src_z = jnp.where(pi_ref[j] == mz, jnp.int32(j), src_z)

        p = 2 * mx + my
        qx = 2 * (1 - mx) + my
        qy = 2 * mx + (1 - my)
        qd = 2 * (1 - mx) + (1 - my)
        xnbr = (1 - mx, my, mz)
        ynbr = (mx, 1 - my, mz)

        barrier = pltpu.get_barrier_semaphore()
        for dev in [(mx, my, src_z), xnbr, ynbr]:
            pl.semaphore_signal(
                barrier, inc=1, device_id=dev,
                device_id_type=pl.DeviceIdType.MESH,
            )
        pl.semaphore_wait(barrier, 3)

        def copy(src, dst, ssem, rsem, dev):
            return pltpu.make_async_remote_copy(
                src_ref=src, dst_ref=dst, send_sem=ssem, recv_sem=rsem,
                device_id=dev, device_id_type=pl.DeviceIdType.MESH,
            )

        absmax = jnp.maximum(
            jnp.max(jnp.abs(x_ref[:, pl.ds(p * q, q), :])), 1e-30
        )
        sc_send[...] = jnp.full((8, 128), absmax * (1.0 / 127.0), jnp.float32)
        rscz = copy(sc_send, scz_recv, sc_sems.at[0], sc_sems.at[1],
                    (mx, my, dst_z))
        rscz.start()

        def chunk(i, c):
            return qbuf.at[:, pl.ds(i * q + c * rc, rc), :]

        inv = 127.0 / absmax
        rz = []
        for c in range(C):
            rows = pl.ds(p * q + c * rc, rc)
            crows = pl.ds(c * rc, rc)
            xq = x_ref[:, rows, :] * inv
            qsend[:, crows, :] = jnp.clip(
                jnp.round(xq), -127.0, 127.0
            ).astype(jnp.int8)
            r = copy(
                qsend.at[:, crows, :], chunk(p, c),
                send_z.at[c], recv_z.at[c], (mx, my, dst_z),
            )
            r.start()
            rz.append(r)

        rscz.wait_recv()
        dq_p = scz_recv[0, 0]
        rscax = copy(scz_recv, scax_recv, sc_sems.at[2], sc_sems.at[3], xnbr)
        rscay = copy(scz_recv, scay_recv, sc_sems.at[4], sc_sems.at[5], ynbr)
        rscax.start()
        rscay.start()

        def dequant(i, c, dq):
            rows = pl.ds(i * q + c * rc, rc)
            out_ref[:, rows, :] = (
                qbuf[:, rows, :].astype(jnp.bfloat16) * dq.astype(jnp.bfloat16)
            )

        rax, ray = [], []
        for c in range(C):
            rz[c].wait_recv()
            ra = copy(chunk(p, c), chunk(p, c), send_ax.at[c], recv_ax.at[c], xnbr)
            ry = copy(chunk(p, c), chunk(p, c), send_ay.at[c], recv_ay.at[c], ynbr)
            ra.start()
            ry.start()
            rax.append(ra)
            ray.append(ry)
            dequant(p, c, dq_p)

        rscax.wait_recv()
        dq_qx = scax_recv[0, 0]
        rscb = copy(scax_recv, scb_recv, sc_sems.at[6], sc_sems.at[7], ynbr)
        rscb.start()
        rscay.wait_recv()
        dq_qy = scay_recv[0, 0]

        rb = []
        for c in range(C):
            if c % 2 == 0:
                rax[c].wait_recv()
                r = copy(chunk(qx, c), chunk(qx, c), send_b.at[c], recv_b.at[c], ynbr)
                r.start()
                dequant(qx, c, dq_qx)
            else:
                ray[c].wait_recv()
                r = copy(chunk(qy, c), chunk(qy, c), send_b.at[c], recv_b.at[c], xnbr)
                r.start()
                dequant(qy, c, dq_qy)
            rb.append(r)

        rscb.wait_recv()
        dq_qd = scb_recv[0, 0]

        for c in range(C):
            if c % 2 == 0:
                ray[c].wait_recv()
                dequant(qy, c, dq_qy)
            else:
                rax[c].wait_recv()
                dequant(qx, c, dq_qx)
            rb[c].wait_recv()
            dequant(qd, c, dq_qd)

        for r in [rscz, rscax, rscay, rscb] + rz + rax + ray + rb:
            r.wait_send()

    return pl.pallas_call(
        body,
        out_shape=jax.ShapeDtypeStruct((1, m, n), jnp.bfloat16),
        in_specs=[
            pl.BlockSpec(memory_space=pltpu.SMEM),
            pl.BlockSpec(memory_space=pltpu.VMEM),
        ],
        out_specs=pl.BlockSpec(memory_space=pltpu.VMEM),
        scratch_shapes=[
            pltpu.VMEM((1, q, n), jnp.int8),
            pltpu.VMEM((1, m, n), jnp.int8),
            pltpu.VMEM((8, 128), jnp.float32),
            pltpu.VMEM((8, 128), jnp.float32),
            pltpu.VMEM((8, 128), jnp.float32),
            pltpu.VMEM((8, 128), jnp.float32),
            pltpu.VMEM((8, 128), jnp.float32),
            pltpu.SemaphoreType.DMA((8,)),
            pltpu.SemaphoreType.DMA((C,)),
            pltpu.SemaphoreType.DMA((C,)),
            pltpu.SemaphoreType.DMA((C,)),
            pltpu.SemaphoreType.DMA((C,)),
            pltpu.SemaphoreType.DMA((C,)),
            pltpu.SemaphoreType.DMA((C,)),
            pltpu.SemaphoreType.DMA((C,)),
            pltpu.SemaphoreType.DMA((C,)),
        ],
        compiler_params=pltpu.CompilerParams(collective_id=0),
    )(pi, x)


# device time: 30062 ns/iter; 1.0716x vs baseline; 1.0097x over previous
import jax
import jax.numpy as jnp
from jax import lax
from jax.experimental import pallas as pl
from jax.experimental.pallas import tpu as pltpu

Z = 4
C = 8


def kernel(x, pi):
    _, m, n = x.shape
    q = m // 4
    rc = q // C

    def body(
        pi_ref, x_ref, out_ref, qsend, qbuf,
        sc_send, scz_recv, scax_recv, scay_recv, scb_recv,
        sc_sems,
        send_z, recv_z, send_ax, recv_ax,
        send_ay, recv_ay, send_b, recv_b,
    ):
        mx = lax.axis_index("x")
        my = lax.axis_index("y")
        mz = lax.axis_index("z")

        dst_z = pi_ref[mz]
        src_z = jnp.int32(0)
        for j in range(Z):
            src_z = jnp.where(pi_ref[j] == mz, jnp.int32(j), src_z)

        p = 2 * mx + my
        qx = 2 * (1 - mx) + my
        qy = 2 * mx + (1 - my)
        qd = 2 * (1 - mx) + (1 - my)
        xnbr = (1 - mx, my, mz)
        ynbr = (mx, 1 - my, mz)

        barrier = pltpu.get_barrier_semaphore()
        for dev in [(mx, my, src_z), xnbr, ynbr]:
            pl.semaphore_signal(
                barrier, inc=1, device_id=dev,
                device_id_type=pl.DeviceIdType.MESH,
            )
        pl.semaphore_wait(barrier, 3)

        def copy(src, dst, ssem, rsem, dev):
            return pltpu.make_async_remote_copy(
                src_ref=src, dst_ref=dst, send_sem=ssem, recv_sem=rsem,
                device_id=dev, device_id_type=pl.DeviceIdType.MESH,
            )

        absmax = jnp.maximum(
            jnp.max(jnp.abs(x_ref[:, pl.ds(p * q, q), :])), 1e-30
        )
        sc_send[...] = jnp.full((8, 128), absmax * (1.0 / 127.0), jnp.float32)
        rscz = copy(sc_send, scz_recv, sc_sems.at[0], sc_sems.at[1],
                    (mx, my, dst_z))
        rscz.start()

        def chunk(i, c):
            return qbuf.at[:, pl.ds(i * q + c * rc, rc), :]

        inv = 127.0 / absmax
        rz = []
        for c in range(C):
            rows = pl.ds(p * q + c * rc, rc)
            crows = pl.ds(c * rc, rc)
            xq = x_ref[:, rows, :] * inv
            qsend[:, crows, :] = jnp.clip(
                jnp.round(xq), -127.0, 127.0
            ).astype(jnp.int8)
            r = copy(
                qsend.at[:, crows, :], chunk(p, c),
                send_z.at[c], recv_z.at[c], (mx, my, dst_z),
            )
            r.start()
            rz.append(r)

        rscz.wait_recv()
        dq_p = scz_recv[0, 0]
        rscax = copy(scz_recv, scax_recv, sc_sems.at[2], sc_sems.at[3], xnbr)
        rscay = copy(scz_recv, scay_recv, sc_sems.at[4], sc_sems.at[5], ynbr)
        rscax.start()
        rscay.start()

        def dequant(i, c, dq):
            rows = pl.ds(i * q + c * rc, rc)
            out_ref[:, rows, :] = (
                qbuf[:, rows, :].astype(jnp.bfloat16) * dq.astype(jnp.bfloat16)
            )

        rax, ray = [], []
        for c in range(C):
            rz[c].wait_recv()
            ra = copy(chunk(p, c), chunk(p, c), send_ax.at[c], recv_ax.at[c], xnbr)
            ry = copy(chunk(p, c), chunk(p, c), send_ay.at[c], recv_ay.at[c], ynbr)
            ra.start()
            ry.start()
            rax.append(ra)
            ray.append(ry)
            dequant(p, c, dq_p)

        rscax.wait_recv()
        dq_qx = scax_recv[0, 0]
        rscb = copy(scax_recv, scb_recv, sc_sems.at[6], sc_sems.at[7], ynbr)
        rscb.start()
        rscay.wait_recv()
        dq_qy = scay_recv[0, 0]

        rb = []
        for c in range(C):
            if c % 2 == 0:
                rax[c].wait_recv()
                r = copy(chunk(qx, c), chunk(qx, c), send_b.at[c], recv_b.at[c], ynbr)
                r.start()
                dequant(qx, c, dq_qx)
            else:
                ray[c].wait_recv()
                r = copy(chunk(qy, c), chunk(qy, c), send_b.at[c], recv_b.at[c], xnbr)
                r.start()
                dequant(qy, c, dq_qy)
            rb.append(r)

        rscb.wait_recv()
        dq_qd = scb_recv[0, 0]

        for c in range(C):
            if c % 2 == 0:
                ray[c].wait_recv()
                dequant(qy, c, dq_qy)
            else:
                rax[c].wait_recv()
                dequant(qx, c, dq_qx)
            rb[c].wait_recv()
            dequant(qd, c, dq_qd)

        for r in [rscz, rscax, rscay, rscb] + rz + rax + ray + rb:
            r.wait_send()

    return pl.pallas_call(
        body,
        out_shape=jax.ShapeDtypeStruct((1, m, n), jnp.bfloat16),
        in_specs=[
            pl.BlockSpec(memory_space=pltpu.SMEM),
            pl.BlockSpec(memory_space=pltpu.VMEM),
        ],
        out_specs=pl.BlockSpec(memory_space=pltpu.VMEM),
        scratch_shapes=[
            pltpu.VMEM((1, q, n), jnp.int8),
            pltpu.VMEM((1, m, n), jnp.int8),
            pltpu.VMEM((8, 128), jnp.float32),
            pltpu.VMEM((8, 128), jnp.float32),
            pltpu.VMEM((8, 128), jnp.float32),
            pltpu.VMEM((8, 128), jnp.float32),
            pltpu.VMEM((8, 128), jnp.float32),
            pltpu.SemaphoreType.DMA((8,)),
            pltpu.SemaphoreType.DMA((C,)),
            pltpu.SemaphoreType.DMA((C,)),
            pltpu.SemaphoreType.DMA((C,)),
            pltpu.SemaphoreType.DMA((C,)),
            pltpu.SemaphoreType.DMA((C,)),
            pltpu.SemaphoreType.DMA((C,)),
            pltpu.SemaphoreType.DMA((C,)),
            pltpu.SemaphoreType.DMA((C,)),
        ],
        compiler_params=pltpu.CompilerParams(collective_id=0),
    )(pi, x)
